# baseline (speedup 1.0000x reference)
import functools

import jax
import jax.numpy as jnp
from jax import lax
from jax.experimental import pallas as pl
from jax.experimental.pallas import tpu as pltpu
from jax.experimental.pallas import tpu_sc as plsc

N_ATOMS_C = 320000
D = 128
NSEG = 1024
NC = 2
NS = 16
NW = NC * NS
PER_TILE = N_ATOMS_C // NW
CH = 40
NCHUNK = PER_TILE // CH
NBUF = 10
ROWS_PER_TILE = NSEG // NS

_mesh = plsc.VectorSubcoreMesh(core_axis_name="c", subcore_axis_name="s")


@functools.partial(
    pl.kernel,
    mesh=_mesh,
    out_type=jax.ShapeDtypeStruct((NC * NSEG, D), jnp.float32),
    scratch_types=(
        [pltpu.VMEM((CH,), jnp.int32) for _ in range(NBUF)]
        + [pltpu.VMEM((CH, D), jnp.float32) for _ in range(NBUF)]
        + [pltpu.VMEM_SHARED((NSEG, D), jnp.float32)]
        + [pltpu.SemaphoreType.DMA for _ in range(2 * NBUF)]
    ),
)
def _segment_sum_sc(feat_hbm, ids_hbm, out_hbm, *refs):
    ids_bufs = refs[0:NBUF]
    rows_bufs = refs[NBUF:2 * NBUF]
    acc_sh = refs[2 * NBUF]
    sem_i = refs[2 * NBUF + 1:3 * NBUF + 1]
    sem_r = refs[3 * NBUF + 1:4 * NBUF + 1]
    cid = lax.axis_index("c")
    sid = lax.axis_index("s")
    wid = cid * NS + sid
    base_row = wid * PER_TILE

    def start_load(c, b):
        off = base_row + c * CH
        pltpu.make_async_copy(
            ids_hbm.at[pl.ds(off, CH)], ids_bufs[b], sem_i[b]).start()
        pltpu.make_async_copy(
            feat_hbm.at[pl.ds(off, CH)], rows_bufs[b], sem_r[b]).start()

    def wait_load(b):
        pltpu.make_async_copy(
            ids_hbm.at[pl.ds(0, CH)], ids_bufs[b], sem_i[b]).wait()
        pltpu.make_async_copy(
            feat_hbm.at[pl.ds(0, CH)], rows_bufs[b], sem_r[b]).wait()

    zero16 = jnp.zeros((16,), jnp.float32)

    def zero_body(i, carry):
        r = i // (D // 16)
        j = i % (D // 16)
        rows_bufs[0][r, pl.ds(j * 16, 16)] = zero16
        return carry

    lax.fori_loop(0, ROWS_PER_TILE * (D // 16), zero_body, 0)
    pltpu.sync_copy(rows_bufs[0].at[pl.ds(0, ROWS_PER_TILE)],
                    acc_sh.at[pl.ds(sid * ROWS_PER_TILE, ROWS_PER_TILE)])
    plsc.subcore_barrier()

    for b in range(NBUF):
        start_load(b, b)

    def group_body(i, carry):
        g = i * NBUF
        for b in range(NBUF):
            c = g + b
            wait_load(b)
            pltpu.sync_copy(rows_bufs[b], acc_sh.at[ids_bufs[b]], add=True)
            start_load(jnp.minimum(c + NBUF, NCHUNK - 1), b)
        return carry

    lax.fori_loop(0, NCHUNK // NBUF, group_body, 0)
    for b in range(NBUF):
        wait_load(b)
    plsc.subcore_barrier()

    pltpu.sync_copy(
        acc_sh.at[pl.ds(sid * ROWS_PER_TILE, ROWS_PER_TILE)],
        out_hbm.at[pl.ds(cid * NSEG + sid * ROWS_PER_TILE, ROWS_PER_TILE)])


def _add2_body(a_ref, b_ref, o_ref):
    o_ref[...] = a_ref[...] + b_ref[...]


def kernel(atom_features, atom_split):
    ids = atom_split.astype(jnp.int32)
    partial = _segment_sum_sc(atom_features, ids)
    return pl.pallas_call(
        _add2_body,
        out_shape=jax.ShapeDtypeStruct((NSEG, D), jnp.float32),
    )(partial[:NSEG], partial[NSEG:])


# speedup vs baseline: 1.1969x; 1.1969x over previous
import functools

import jax
import jax.numpy as jnp
from jax import lax
from jax.experimental import pallas as pl
from jax.experimental.pallas import tpu as pltpu
from jax.experimental.pallas import tpu_sc as plsc

N_ATOMS_C = 320000
D = 128
NSEG = 1024
NC = 2
NS = 16
NW = NC * NS
PER_TILE = N_ATOMS_C // NW
CH = 128
NCHUNK = PER_TILE // CH
TAIL = PER_TILE - NCHUNK * CH
NBUF = 6
ROWS_PER_TILE = NSEG // NS

_mesh = plsc.VectorSubcoreMesh(core_axis_name="c", subcore_axis_name="s")


@functools.partial(
    pl.kernel,
    mesh=_mesh,
    out_type=jax.ShapeDtypeStruct((NC * NSEG, D), jnp.float32),
    scratch_types=(
        [pltpu.VMEM((CH,), jnp.int32) for _ in range(NBUF)]
        + [pltpu.VMEM((CH, D), jnp.float32) for _ in range(NBUF)]
        + [pltpu.VMEM((TAIL,), jnp.int32)]
        + [pltpu.VMEM_SHARED((NSEG, D), jnp.float32)]
        + [pltpu.SemaphoreType.DMA for _ in range(2 * NBUF)]
    ),
)
def _segment_sum_sc(feat_hbm, ids_hbm, out_hbm, *refs):
    ids_bufs = refs[0:NBUF]
    rows_bufs = refs[NBUF:2 * NBUF]
    ids_tail = refs[2 * NBUF]
    acc_sh = refs[2 * NBUF + 1]
    sem_i = refs[2 * NBUF + 2:3 * NBUF + 2]
    sem_r = refs[3 * NBUF + 2:4 * NBUF + 2]
    cid = lax.axis_index("c")
    sid = lax.axis_index("s")
    wid = cid * NS + sid
    base_row = wid * PER_TILE

    def start_load(c, b):
        off = base_row + c * CH
        pltpu.make_async_copy(
            ids_hbm.at[pl.ds(off, CH)], ids_bufs[b], sem_i[b]).start()
        pltpu.make_async_copy(
            feat_hbm.at[pl.ds(off, CH)], rows_bufs[b], sem_r[b]).start()

    def wait_load(b):
        pltpu.make_async_copy(
            ids_hbm.at[pl.ds(0, CH)], ids_bufs[b], sem_i[b]).wait()
        pltpu.make_async_copy(
            feat_hbm.at[pl.ds(0, CH)], rows_bufs[b], sem_r[b]).wait()

    zero16 = jnp.zeros((16,), jnp.float32)

    def zero_body(i, carry):
        r = i // (D // 16)
        j = i % (D // 16)
        rows_bufs[0][r, pl.ds(j * 16, 16)] = zero16
        return carry

    lax.fori_loop(0, ROWS_PER_TILE * (D // 16), zero_body, 0)
    pltpu.sync_copy(rows_bufs[0].at[pl.ds(0, ROWS_PER_TILE)],
                    acc_sh.at[pl.ds(sid * ROWS_PER_TILE, ROWS_PER_TILE)])
    plsc.subcore_barrier()

    for b in range(NBUF):
        start_load(b, b)

    def group_body(i, carry):
        g = i * NBUF
        for b in range(NBUF):
            c = g + b
            wait_load(b)
            pltpu.sync_copy(rows_bufs[b], acc_sh.at[ids_bufs[b]], add=True)
            start_load(jnp.minimum(c + NBUF, NCHUNK - 1), b)
        return carry

    lax.fori_loop(0, NCHUNK // NBUF, group_body, 0)
    for b in range(NBUF):
        wait_load(b)

    tail_off = base_row + NCHUNK * CH
    pltpu.sync_copy(ids_hbm.at[pl.ds(tail_off, TAIL)], ids_tail)
    pltpu.sync_copy(feat_hbm.at[pl.ds(tail_off, TAIL)],
                    rows_bufs[0].at[pl.ds(0, TAIL)])
    pltpu.sync_copy(rows_bufs[0].at[pl.ds(0, TAIL)],
                    acc_sh.at[ids_tail], add=True)
    plsc.subcore_barrier()

    pltpu.sync_copy(
        acc_sh.at[pl.ds(sid * ROWS_PER_TILE, ROWS_PER_TILE)],
        out_hbm.at[pl.ds(cid * NSEG + sid * ROWS_PER_TILE, ROWS_PER_TILE)])


def _add2_body(a_ref, b_ref, o_ref):
    o_ref[...] = a_ref[...] + b_ref[...]


def kernel(atom_features, atom_split):
    ids = atom_split.astype(jnp.int32)
    partial = _segment_sum_sc(atom_features, ids)
    return pl.pallas_call(
        _add2_body,
        out_shape=jax.ShapeDtypeStruct((NSEG, D), jnp.float32),
    )(partial[:NSEG], partial[NSEG:])
